# Initial kernel scaffold; baseline (speedup 1.0000x reference)
import jax
import jax.numpy as jnp
from jax.experimental import pallas as pl


def kernel(x, edge_index, edge_weight, batch, W1, b1, W2, b2, Wo, bo, Wb, bb):
    raise NotImplementedError("write your pallas kernel here")


# speedup vs baseline: 17.5277x; 17.5277x over previous
import functools

import jax
import jax.numpy as jnp
from jax import lax
from jax.experimental import pallas as pl
from jax.experimental.pallas import tpu as pltpu
from jax.experimental.pallas import tpu_sc as plsc

N = 100000
E = 3200000
G = 1024
FEAT = 4
HID = 32
HH = 16

NC = 2
NS = 16
ROWPAD = 100352
STRIPE = ROWPAD // NS
ZROWS = 784
CH = 128
BR = 3136

_mesh = plsc.VectorSubcoreMesh(core_axis_name="c", subcore_axis_name="s")
_sc_params = pltpu.CompilerParams(use_tc_tiling_on_sc=False,
                                  needs_layout_passes=False)


def _zero_vec_loop(ref, nvec):
  def zf(j, _):
    ref[pl.ds(j * 16, 16)] = jnp.zeros((16,), jnp.float32)
    return 0
  lax.fori_loop(0, nvec, zf, 0)


def _k1_body(dst_hbm, ew_hbm, batch_hbm, deg_out, cnt_out,
             idx_v, upd_v, zb_v, deg_sh, cnt_sh, sem_i, sem_w):
  c = lax.axis_index("c")
  s = lax.axis_index("s")
  _zero_vec_loop(zb_v, STRIPE // 16)
  pltpu.sync_copy(zb_v, deg_sh.at[pl.ds(s * STRIPE, STRIPE)])

  @pl.when(s == 0)
  def _():
    pltpu.sync_copy(zb_v.at[pl.ds(0, G)], cnt_sh)

  plsc.subcore_barrier()

  nch = (E // 2) // CH
  my_n = (nch - s + NS - 1) // NS

  def dbody(i, _):
    q = s + i * NS
    base = c * (E // 2) + q * CH
    cp1 = pltpu.async_copy(dst_hbm.at[pl.ds(base, CH)], idx_v, sem_i)
    cp2 = pltpu.async_copy(ew_hbm.at[pl.ds(base, CH)], upd_v, sem_w)
    cp1.wait()
    cp2.wait()
    pltpu.sync_copy(upd_v, deg_sh.at[idx_v], add=True)
    return 0

  lax.fori_loop(0, my_n, dbody, 0)

  nchn = (ROWPAD // 2) // CH
  my_m = (nchn - s + NS - 1) // NS

  def cbody(i, _):
    q = s + i * NS
    base = c * (ROWPAD // 2) + q * CH
    cp = pltpu.async_copy(batch_hbm.at[pl.ds(base, CH)], idx_v, sem_i)

    def wf(j, _):
      node = base + j * 16 + lax.iota(jnp.int32, 16)
      upd_v[pl.ds(j * 16, 16)] = jnp.where(node < N, 1.0, 0.0)
      return 0

    lax.fori_loop(0, CH // 16, wf, 0)
    cp.wait()
    pltpu.sync_copy(upd_v, cnt_sh.at[idx_v], add=True)
    return 0

  lax.fori_loop(0, my_m, cbody, 0)
  plsc.subcore_barrier()
  pltpu.sync_copy(deg_sh.at[pl.ds(s * STRIPE, STRIPE)],
                  deg_out.at[pl.ds(c * ROWPAD + s * STRIPE, STRIPE)])

  @pl.when(s == 0)
  def _():
    pltpu.sync_copy(cnt_sh, cnt_out.at[pl.ds(c * G, G)])


_k1 = pl.kernel(
    _k1_body,
    out_type=[jax.ShapeDtypeStruct((2 * ROWPAD,), jnp.float32),
              jax.ShapeDtypeStruct((2 * G,), jnp.float32)],
    mesh=_mesh,
    compiler_params=_sc_params,
    scratch_types=[
        pltpu.VMEM((CH,), jnp.int32),
        pltpu.VMEM((CH,), jnp.float32),
        pltpu.VMEM((STRIPE,), jnp.float32),
        pltpu.VMEM_SHARED((ROWPAD,), jnp.float32),
        pltpu.VMEM_SHARED((G,), jnp.float32),
        pltpu.SemaphoreType.DMA,
        pltpu.SemaphoreType.DMA,
    ],
)


def _edge_scatter_body(epc, ebase_stride, toff,
                       tab_hbm, src_hbm, dst_hbm, ew_hbm, out_hbm,
                       idx_v, di_v, ew_v, rows_v, zb_v, acc_sh,
                       sem_i, sem_d, sem_w, sem_g):
  c = lax.axis_index("c")
  s = lax.axis_index("s")

  def zf(j, _):
    zb_v[j, :] = jnp.zeros((16,), jnp.float32)
    return 0

  lax.fori_loop(0, ZROWS, zf, 0)
  for k in range(STRIPE // ZROWS):
    pltpu.sync_copy(zb_v, acc_sh.at[pl.ds(s * STRIPE + k * ZROWS, ZROWS)])
  plsc.subcore_barrier()

  nch = epc // CH
  my_n = (nch - s + NS - 1) // NS

  def body(i, _):
    q = s + i * NS
    base = c * ebase_stride + q * CH
    cp1 = pltpu.async_copy(src_hbm.at[pl.ds(base, CH)], idx_v, sem_i)
    cp2 = pltpu.async_copy(dst_hbm.at[pl.ds(base, CH)], di_v, sem_d)
    cp3 = pltpu.async_copy(ew_hbm.at[pl.ds(base, CH)], ew_v, sem_w)
    cp1.wait()
    if toff:
      off = c * toff

      def af(j, _):
        idx_v[pl.ds(j * 16, 16)] = idx_v[pl.ds(j * 16, 16)] + off
        return 0

      lax.fori_loop(0, CH // 16, af, 0)
    pltpu.async_copy(tab_hbm.at[idx_v], rows_v, sem_g).wait()
    cp3.wait()

    def mf(j, _):
      ew16 = ew_v[pl.ds(j * 16, 16)]
      for l in range(16):
        r = j * 16 + l
        rows_v[r, :] = rows_v[r, :] * ew16[l]
      return 0

    lax.fori_loop(0, CH // 16, mf, 0)
    cp2.wait()
    pltpu.sync_copy(rows_v, acc_sh.at[di_v], add=True)
    return 0

  lax.fori_loop(0, my_n, body, 0)
  plsc.subcore_barrier()
  pltpu.sync_copy(acc_sh.at[pl.ds(s * STRIPE, STRIPE)],
                  out_hbm.at[pl.ds(c * ROWPAD + s * STRIPE, STRIPE)])


def _make_edge_scatter(epc, ebase_stride, toff):
  return pl.kernel(
      functools.partial(_edge_scatter_body, epc, ebase_stride, toff),
      out_type=jax.ShapeDtypeStruct((2 * ROWPAD, HH), jnp.float32),
      mesh=_mesh,
      compiler_params=_sc_params,
      scratch_types=[
          pltpu.VMEM((CH,), jnp.int32),
          pltpu.VMEM((CH,), jnp.int32),
          pltpu.VMEM((CH,), jnp.float32),
          pltpu.VMEM((CH, HH), jnp.float32),
          pltpu.VMEM((ZROWS, HH), jnp.float32),
          pltpu.VMEM_SHARED((ROWPAD, HH), jnp.float32),
          pltpu.SemaphoreType.DMA,
          pltpu.SemaphoreType.DMA,
          pltpu.SemaphoreType.DMA,
          pltpu.SemaphoreType.DMA,
      ],
  )


_k3 = _make_edge_scatter(E // 2, E // 2, 0)
_k5 = _make_edge_scatter(E, 0, ROWPAD)


def _k7_body(h2_hbm, batch_hbm, out_hbm, rows_v, b_v, acc_v, sem_r, sem_b):
  c = lax.axis_index("c")
  s = lax.axis_index("s")
  wid = s * NC + c

  def zf(r, _):
    acc_v[r, 0:16] = jnp.zeros((16,), jnp.float32)
    acc_v[r, 16:32] = jnp.zeros((16,), jnp.float32)
    return 0

  lax.fori_loop(0, G, zf, 0)

  npt = ROWPAD // 32
  cs = npt // 4
  col = lax.iota(jnp.int32, 16)
  for k in range(4):
    base = wid * npt + k * cs
    cp = pltpu.async_copy(h2_hbm.at[pl.ds(base, cs)], rows_v, sem_r)
    cb = pltpu.async_copy(batch_hbm.at[pl.ds(base, cs)], b_v, sem_b)
    cp.wait()
    cb.wait()

    def nf(j, _):
      b16 = b_v[pl.ds(j * 16, 16)]
      for l in range(16):
        r = j * 16 + l
        valid = (base + r) < N
        m = jnp.full((16,), valid, jnp.bool_)
        ri = jnp.full((16,), b16[l], jnp.int32)
        plsc.addupdate_scatter(acc_v, [ri, col], rows_v[r, 0:16], mask=m)
        plsc.addupdate_scatter(acc_v, [ri, col + 16], rows_v[r, 16:32], mask=m)
      return 0

    lax.fori_loop(0, cs // 16, nf, 0)
  pltpu.sync_copy(acc_v, out_hbm.at[wid])


_k7 = pl.kernel(
    _k7_body,
    out_type=jax.ShapeDtypeStruct((32, G, HID), jnp.float32),
    mesh=_mesh,
    compiler_params=_sc_params,
    scratch_types=[
        pltpu.VMEM((ROWPAD // 32 // 4, HID), jnp.float32),
        pltpu.VMEM((ROWPAD // 32 // 4,), jnp.int32),
        pltpu.VMEM((G, HID), jnp.float32),
        pltpu.SemaphoreType.DMA,
        pltpu.SemaphoreType.DMA,
    ],
)


def _k2_body(deg_ref, x_ref, dinvb_ref, g0_ref):
  deg = deg_ref[0] + deg_ref[1] + 1.0
  dinv = lax.rsqrt(deg)
  dinvb = jnp.broadcast_to(dinv, (BR, HH))
  dinvb_ref[...] = dinvb
  g0_ref[...] = dinvb * x_ref[...]


_k2 = pl.pallas_call(
    _k2_body,
    grid=(ROWPAD // BR,),
    in_specs=[
        pl.BlockSpec((2, BR, 1), lambda i: (0, i, 0)),
        pl.BlockSpec((BR, HH), lambda i: (i, 0)),
    ],
    out_specs=[
        pl.BlockSpec((BR, HH), lambda i: (i, 0)),
        pl.BlockSpec((BR, HH), lambda i: (i, 0)),
    ],
    out_shape=[jax.ShapeDtypeStruct((ROWPAD, HH), jnp.float32),
               jax.ShapeDtypeStruct((ROWPAD, HH), jnp.float32)],
)


def _k4_body(p_ref, g0_ref, dinvb_ref, w_ref, b_ref, out_ref):
  dinvb = dinvb_ref[...]
  agg = dinvb * (p_ref[0] + p_ref[1] + g0_ref[...])
  h = jnp.dot(agg, w_ref[...], preferred_element_type=jnp.float32)
  h = jnp.maximum(h + b_ref[...], 0.0)
  out_ref[0] = dinvb * h[:, 0:HH]
  out_ref[1] = dinvb * h[:, HH:HID]


_k4 = pl.pallas_call(
    _k4_body,
    grid=(ROWPAD // BR,),
    in_specs=[
        pl.BlockSpec((2, BR, HH), lambda i: (0, i, 0)),
        pl.BlockSpec((BR, HH), lambda i: (i, 0)),
        pl.BlockSpec((BR, HH), lambda i: (i, 0)),
        pl.BlockSpec((HH, HID), lambda i: (0, 0)),
        pl.BlockSpec((1, HID), lambda i: (0, 0)),
    ],
    out_specs=pl.BlockSpec((2, BR, HH), lambda i: (0, i, 0)),
    out_shape=jax.ShapeDtypeStruct((2, ROWPAD, HH), jnp.float32),
)


def _k6_body(p_ref, g1_ref, dinvb_ref, w_ref, b_ref, out_ref):
  pa = p_ref[0] + g1_ref[0]
  pb = p_ref[1] + g1_ref[1]
  cat = jnp.concatenate([pa, pb], axis=1)
  dinvb = dinvb_ref[...]
  d2 = jnp.concatenate([dinvb, dinvb], axis=1)
  h = jnp.dot(d2 * cat, w_ref[...], preferred_element_type=jnp.float32)
  out_ref[...] = jnp.maximum(h + b_ref[...], 0.0)


_k6 = pl.pallas_call(
    _k6_body,
    grid=(ROWPAD // BR,),
    in_specs=[
        pl.BlockSpec((2, BR, HH), lambda i: (0, i, 0)),
        pl.BlockSpec((2, BR, HH), lambda i: (0, i, 0)),
        pl.BlockSpec((BR, HH), lambda i: (i, 0)),
        pl.BlockSpec((HID, HID), lambda i: (0, 0)),
        pl.BlockSpec((1, HID), lambda i: (0, 0)),
    ],
    out_specs=pl.BlockSpec((BR, HID), lambda i: (i, 0)),
    out_shape=jax.ShapeDtypeStruct((ROWPAD, HID), jnp.float32),
)


def _k8_body(pool_ref, cnt_ref, wo_ref, bo_ref, wb_ref, bb_ref, o_ref, b_ref):
  sums = jnp.sum(pool_ref[...], axis=0)
  cnt = cnt_ref[0] + cnt_ref[1]
  g = sums / jnp.clip(cnt, 1.0, None)[:, None]
  o_ref[...] = jax.nn.sigmoid(
      jnp.dot(g, wo_ref[...], preferred_element_type=jnp.float32) + bo_ref[...])
  b_ref[...] = jax.nn.sigmoid(
      jnp.dot(g, wb_ref[...], preferred_element_type=jnp.float32) + bb_ref[...])


_k8 = pl.pallas_call(
    _k8_body,
    out_shape=[jax.ShapeDtypeStruct((G, 1), jnp.float32),
               jax.ShapeDtypeStruct((G, 1), jnp.float32)],
)


def kernel(x, edge_index, edge_weight, batch, W1, b1, W2, b2, Wo, bo, Wb, bb):
  src = edge_index[0]
  dst = edge_index[1]
  xpad = jnp.pad(x, ((0, ROWPAD - N), (0, HH - FEAT)))
  batch_pad = jnp.pad(batch, (0, ROWPAD - N))
  w1pad = jnp.pad(W1, ((0, HH - FEAT), (0, 0)))

  deg_flat, cnt_flat = _k1(dst, edge_weight, batch_pad)
  dinvb, g0 = _k2(deg_flat.reshape(2, ROWPAD, 1), xpad)
  p1 = _k3(g0, src, dst, edge_weight)
  g1 = _k4(p1.reshape(2, ROWPAD, HH), g0, dinvb, w1pad, b1.reshape(1, HID))
  p2 = _k5(g1.reshape(2 * ROWPAD, HH), src, dst, edge_weight)
  h2 = _k6(p2.reshape(2, ROWPAD, HH), g1, dinvb, W2, b2.reshape(1, HID))
  pool = _k7(h2, batch_pad)
  orange, blue = _k8(pool, cnt_flat.reshape(2, G), Wo, bo.reshape(1, 1),
                     Wb, bb.reshape(1, 1))
  return (orange, blue)


# speedup vs baseline: 49.0818x; 2.8002x over previous
import functools

import jax
import jax.numpy as jnp
from jax import lax
from jax.experimental import pallas as pl
from jax.experimental.pallas import tpu as pltpu
from jax.experimental.pallas import tpu_sc as plsc

N = 100000
E = 3200000
G = 1024
FEAT = 4
HID = 32
HH = 16

NC = 2
NS = 16
ROWPAD = 100352
STRIPE = ROWPAD // NS
ZROWS = 392
CH = 512
CHR = CH // 128
EPAD = 3211264
ERTOT = EPAD // 128
BR = 3136

_mesh = plsc.VectorSubcoreMesh(core_axis_name="c", subcore_axis_name="s")
_sc_params = pltpu.CompilerParams(use_tc_tiling_on_sc=False,
                                  needs_layout_passes=False)


def _zero_vec_loop(ref, nvec):
  def zf(j, _):
    ref[pl.ds(j * 16, 16)] = jnp.zeros((16,), jnp.float32)
    return 0
  lax.fori_loop(0, nvec, zf, 0)


def _k1_body(dst_hbm, ew_hbm, batch_hbm, deg_out, cnt_out,
             di0, di1, up0, up1, idx_v, w_v, zb_v, deg_sh, cnt_sh,
             sem_d0, sem_d1, sem_w0, sem_w1, sem_c):
  c = lax.axis_index("c")
  s = lax.axis_index("s")
  _zero_vec_loop(zb_v, STRIPE // 16)
  pltpu.sync_copy(zb_v, deg_sh.at[pl.ds(s * STRIPE, STRIPE)])

  @pl.when(s == 0)
  def _():
    pltpu.sync_copy(zb_v.at[pl.ds(0, G)], cnt_sh)

  plsc.subcore_barrier()

  ncht = (EPAD // 2) // CH // NS
  dbuf = (di0, di1)
  ubuf = (up0, up1)
  dsem = (sem_d0, sem_d1)
  usem = (sem_w0, sem_w1)

  def ebase(k):
    q = s + k * NS
    return (c * ((EPAD // 2) // CH) + q) * CH

  def issue_in(k, p):
    eb = ebase(k)
    pltpu.async_copy(dst_hbm.at[pl.ds(eb, CH)], dbuf[p], dsem[p])
    pltpu.async_copy(ew_hbm.at[pl.ds(eb, CH)], ubuf[p], usem[p])

  def wait_in(p):
    pltpu.make_async_copy(dst_hbm.at[pl.ds(0, CH)], dbuf[p], dsem[p]).wait()
    pltpu.make_async_copy(ew_hbm.at[pl.ds(0, CH)], ubuf[p], usem[p]).wait()

  def do_scatter(p):
    pltpu.sync_copy(ubuf[p], deg_sh.at[dbuf[p]], add=True)

  issue_in(0, 0)

  def dpair(k2, _):
    k = 2 * k2
    issue_in(k + 1, 1)
    wait_in(0)
    do_scatter(0)
    issue_in(k + 2, 0)
    wait_in(1)
    do_scatter(1)
    return 0

  lax.fori_loop(0, (ncht - 2) // 2, dpair, 0)
  issue_in(ncht - 1, 1)
  wait_in(0)
  do_scatter(0)
  wait_in(1)
  do_scatter(1)

  nchn = (ROWPAD // 2) // 128
  my_m = (nchn - s + NS - 1) // NS

  def cbody(i, _):
    q = s + i * NS
    base = c * (ROWPAD // 2) + q * 128
    cp = pltpu.async_copy(batch_hbm.at[pl.ds(base, 128)], idx_v, sem_c)

    def wf(j, _):
      node = base + j * 16 + lax.iota(jnp.int32, 16)
      w_v[pl.ds(j * 16, 16)] = jnp.where(node < N, 1.0, 0.0)
      return 0

    lax.fori_loop(0, 8, wf, 0)
    cp.wait()
    pltpu.sync_copy(w_v, cnt_sh.at[idx_v], add=True)
    return 0

  lax.fori_loop(0, my_m, cbody, 0)
  plsc.subcore_barrier()
  pltpu.sync_copy(deg_sh.at[pl.ds(s * STRIPE, STRIPE)],
                  deg_out.at[pl.ds(c * ROWPAD + s * STRIPE, STRIPE)])

  @pl.when(s == 0)
  def _():
    pltpu.sync_copy(cnt_sh, cnt_out.at[pl.ds(c * G, G)])


_k1 = pl.kernel(
    _k1_body,
    out_type=[jax.ShapeDtypeStruct((2 * ROWPAD,), jnp.float32),
              jax.ShapeDtypeStruct((2 * G,), jnp.float32)],
    mesh=_mesh,
    compiler_params=_sc_params,
    scratch_types=[
        pltpu.VMEM((CH,), jnp.int32),
        pltpu.VMEM((CH,), jnp.int32),
        pltpu.VMEM((CH,), jnp.float32),
        pltpu.VMEM((CH,), jnp.float32),
        pltpu.VMEM((128,), jnp.int32),
        pltpu.VMEM((128,), jnp.float32),
        pltpu.VMEM((STRIPE,), jnp.float32),
        pltpu.VMEM_SHARED((ROWPAD,), jnp.float32),
        pltpu.VMEM_SHARED((G,), jnp.float32),
        pltpu.SemaphoreType.DMA,
        pltpu.SemaphoreType.DMA,
        pltpu.SemaphoreType.DMA,
        pltpu.SemaphoreType.DMA,
        pltpu.SemaphoreType.DMA,
    ],
)


def _edge_scatter_body(nchc, toff,
                       tab_hbm, src_hbm, dst_hbm, ew_hbm, out_hbm,
                       ix0, ix1, di0, di1, ew0, ew1, rw0, rw1, zb_v, acc_sh,
                       sem_s0, sem_s1, sem_i0, sem_i1, sem_g0, sem_g1):
  c = lax.axis_index("c")
  s = lax.axis_index("s")

  def zf(j, _):
    zb_v[j, :] = jnp.zeros((16,), jnp.float32)
    return 0

  lax.fori_loop(0, ZROWS, zf, 0)
  for k in range(STRIPE // ZROWS):
    pltpu.sync_copy(zb_v, acc_sh.at[pl.ds(s * STRIPE + k * ZROWS, ZROWS)])
  plsc.subcore_barrier()

  ncht = nchc // NS
  ixb = (ix0, ix1)
  dib = (di0, di1)
  ewb = (ew0, ew1)
  rwb = (rw0, rw1)
  ssem = (sem_s0, sem_s1)
  isem = (sem_i0, sem_i1)
  gsem = (sem_g0, sem_g1)

  def ebase(k):
    q = s + k * NS
    return (c * (toff == 0) * nchc + q) * CH

  def issue_src(k, p):
    pltpu.async_copy(src_hbm.at[pl.ds(ebase(k), CH)], ixb[p], ssem[p])

  def wait_src(p):
    pltpu.make_async_copy(src_hbm.at[pl.ds(0, CH)], ixb[p], ssem[p]).wait()

  def issue_dw(k, p):
    eb = ebase(k)
    pltpu.async_copy(dst_hbm.at[pl.ds(eb, CH)], dib[p], isem[p])
    pltpu.async_copy(ew_hbm.at[pl.ds(eb, CH)], ewb[p], isem[p])

  def wait_dw(p):
    pltpu.make_async_copy(dst_hbm.at[pl.ds(0, CH)], dib[p], isem[p]).wait()
    pltpu.make_async_copy(ew_hbm.at[pl.ds(0, CH)], ewb[p], isem[p]).wait()

  def issue_gather(p):
    if toff:
      off = c * toff

      def af(j, _):
        ixb[p][pl.ds(j * 16, 16)] = ixb[p][pl.ds(j * 16, 16)] + off
        return 0

      lax.fori_loop(0, CH // 16, af, 0)
    pltpu.async_copy(tab_hbm.at[ixb[p]], rwb[p], gsem[p])

  def wait_gather(p):
    pltpu.make_async_copy(tab_hbm.at[ixb[p]], rwb[p], gsem[p]).wait()

  def scale(p):
    rows = rwb[p]
    ew = ewb[p]

    def sg(gi, _):
      ew16 = ew[pl.ds(gi * 16, 16)]
      for l in range(16):
        r = gi * 16 + l
        rows[r, :] = rows[r, :] * ew16[l]
      return 0

    lax.fori_loop(0, CH // 16, sg, 0)

  def do_scatter(p):
    pltpu.sync_copy(rwb[p], acc_sh.at[dib[p]], add=True)

  issue_src(0, 0)
  issue_src(1, 1)
  issue_dw(0, 0)
  wait_src(0)
  issue_gather(0)

  def step(k, p0, has_next, has_next2):
    p1 = 1 - p0
    if has_next:
      wait_src(p1)
      issue_gather(p1)
      issue_dw(k + 1, p1)
    wait_gather(p0)
    wait_dw(p0)
    scale(p0)
    if has_next2:
      issue_src(k + 2, p0)
    do_scatter(p0)

  step(0, 0, True, True)

  def pair(k2, _):
    k = 2 * k2 + 1
    step(k, 1, True, True)
    step(k + 1, 0, True, True)
    return 0

  lax.fori_loop(0, (ncht - 4) // 2, pair, 0)
  step(ncht - 3, 1, True, True)
  step(ncht - 2, 0, True, False)
  step(ncht - 1, 1, False, False)

  plsc.subcore_barrier()
  pltpu.sync_copy(acc_sh.at[pl.ds(s * STRIPE, STRIPE)],
                  out_hbm.at[pl.ds(c * ROWPAD + s * STRIPE, STRIPE)])


def _make_edge_scatter(nchc, toff):
  return pl.kernel(
      functools.partial(_edge_scatter_body, nchc, toff),
      out_type=jax.ShapeDtypeStruct((2 * ROWPAD, HH), jnp.float32),
      mesh=_mesh,
      compiler_params=_sc_params,
      scratch_types=[
          pltpu.VMEM((CH,), jnp.int32),
          pltpu.VMEM((CH,), jnp.int32),
          pltpu.VMEM((CH,), jnp.int32),
          pltpu.VMEM((CH,), jnp.int32),
          pltpu.VMEM((CH,), jnp.float32),
          pltpu.VMEM((CH,), jnp.float32),
          pltpu.VMEM((CH, HH), jnp.float32),
          pltpu.VMEM((CH, HH), jnp.float32),
          pltpu.VMEM((ZROWS, HH), jnp.float32),
          pltpu.VMEM_SHARED((ROWPAD, HH), jnp.float32),
          pltpu.SemaphoreType.DMA,
          pltpu.SemaphoreType.DMA,
          pltpu.SemaphoreType.DMA,
          pltpu.SemaphoreType.DMA,
          pltpu.SemaphoreType.DMA,
          pltpu.SemaphoreType.DMA,
      ],
  )


_k3 = _make_edge_scatter((EPAD // 2) // CH, 0)
_k5 = _make_edge_scatter(EPAD // CH, ROWPAD)


def _k7_body(h2_hbm, batch_hbm, out_hbm, rows_v, b_v, acc_v, sem_r, sem_b):
  c = lax.axis_index("c")
  s = lax.axis_index("s")
  wid = s * NC + c

  def zf(r, _):
    acc_v[r, 0:16] = jnp.zeros((16,), jnp.float32)
    acc_v[r, 16:32] = jnp.zeros((16,), jnp.float32)
    return 0

  lax.fori_loop(0, G, zf, 0)

  npt = ROWPAD // 32
  cs = npt // 4
  col = lax.iota(jnp.int32, 16)
  for k in range(4):
    base = wid * npt + k * cs
    cp = pltpu.async_copy(h2_hbm.at[pl.ds(base, cs)], rows_v, sem_r)
    cb = pltpu.async_copy(batch_hbm.at[pl.ds(base, cs)], b_v, sem_b)
    cp.wait()
    cb.wait()

    def nf(j, _):
      b16 = b_v[pl.ds(j * 16, 16)]
      for l in range(16):
        r = j * 16 + l
        valid = (base + r) < N
        m = jnp.full((16,), valid, jnp.bool_)
        ri = jnp.full((16,), b16[l], jnp.int32)
        plsc.addupdate_scatter(acc_v, [ri, col], rows_v[r, 0:16], mask=m)
        plsc.addupdate_scatter(acc_v, [ri, col + 16], rows_v[r, 16:32], mask=m)
      return 0

    lax.fori_loop(0, cs // 16, nf, 0)
  pltpu.sync_copy(acc_v, out_hbm.at[wid])


_k7 = pl.kernel(
    _k7_body,
    out_type=jax.ShapeDtypeStruct((32, G, HID), jnp.float32),
    mesh=_mesh,
    compiler_params=_sc_params,
    scratch_types=[
        pltpu.VMEM((ROWPAD // 32 // 4, HID), jnp.float32),
        pltpu.VMEM((ROWPAD // 32 // 4,), jnp.int32),
        pltpu.VMEM((G, HID), jnp.float32),
        pltpu.SemaphoreType.DMA,
        pltpu.SemaphoreType.DMA,
    ],
)


def _k2_body(deg_ref, x_ref, dinvb_ref, g0_ref):
  deg = deg_ref[0] + deg_ref[1] + 1.0
  dinv = lax.rsqrt(deg)
  dinvb = jnp.broadcast_to(dinv, (BR, HH))
  dinvb_ref[...] = dinvb
  g0_ref[...] = dinvb * x_ref[...]


_k2 = pl.pallas_call(
    _k2_body,
    grid=(ROWPAD // BR,),
    in_specs=[
        pl.BlockSpec((2, BR, 1), lambda i: (0, i, 0)),
        pl.BlockSpec((BR, HH), lambda i: (i, 0)),
    ],
    out_specs=[
        pl.BlockSpec((BR, HH), lambda i: (i, 0)),
        pl.BlockSpec((BR, HH), lambda i: (i, 0)),
    ],
    out_shape=[jax.ShapeDtypeStruct((ROWPAD, HH), jnp.float32),
               jax.ShapeDtypeStruct((ROWPAD, HH), jnp.float32)],
)


def _k4_body(p_ref, g0_ref, dinvb_ref, w_ref, b_ref, out_ref):
  dinvb = dinvb_ref[...]
  agg = dinvb * (p_ref[0] + p_ref[1] + g0_ref[...])
  h = jnp.dot(agg, w_ref[...], preferred_element_type=jnp.float32)
  h = jnp.maximum(h + b_ref[...], 0.0)
  out_ref[0] = dinvb * h[:, 0:HH]
  out_ref[1] = dinvb * h[:, HH:HID]


_k4 = pl.pallas_call(
    _k4_body,
    grid=(ROWPAD // BR,),
    in_specs=[
        pl.BlockSpec((2, BR, HH), lambda i: (0, i, 0)),
        pl.BlockSpec((BR, HH), lambda i: (i, 0)),
        pl.BlockSpec((BR, HH), lambda i: (i, 0)),
        pl.BlockSpec((HH, HID), lambda i: (0, 0)),
        pl.BlockSpec((1, HID), lambda i: (0, 0)),
    ],
    out_specs=pl.BlockSpec((2, BR, HH), lambda i: (0, i, 0)),
    out_shape=jax.ShapeDtypeStruct((2, ROWPAD, HH), jnp.float32),
)


def _k6_body(p_ref, g1_ref, dinvb_ref, w_ref, b_ref, out_ref):
  pa = p_ref[0] + g1_ref[0]
  pb = p_ref[1] + g1_ref[1]
  cat = jnp.concatenate([pa, pb], axis=1)
  dinvb = dinvb_ref[...]
  d2 = jnp.concatenate([dinvb, dinvb], axis=1)
  h = jnp.dot(d2 * cat, w_ref[...], preferred_element_type=jnp.float32)
  out_ref[...] = jnp.maximum(h + b_ref[...], 0.0)


_k6 = pl.pallas_call(
    _k6_body,
    grid=(ROWPAD // BR,),
    in_specs=[
        pl.BlockSpec((2, BR, HH), lambda i: (0, i, 0)),
        pl.BlockSpec((2, BR, HH), lambda i: (0, i, 0)),
        pl.BlockSpec((BR, HH), lambda i: (i, 0)),
        pl.BlockSpec((HID, HID), lambda i: (0, 0)),
        pl.BlockSpec((1, HID), lambda i: (0, 0)),
    ],
    out_specs=pl.BlockSpec((BR, HID), lambda i: (i, 0)),
    out_shape=jax.ShapeDtypeStruct((ROWPAD, HID), jnp.float32),
)


def _k8_body(pool_ref, cnt_ref, wo_ref, bo_ref, wb_ref, bb_ref, o_ref, b_ref):
  sums = jnp.sum(pool_ref[...], axis=0)
  cnt = cnt_ref[0] + cnt_ref[1]
  g = sums / jnp.clip(cnt, 1.0, None)[:, None]
  o_ref[...] = jax.nn.sigmoid(
      jnp.dot(g, wo_ref[...], preferred_element_type=jnp.float32) + bo_ref[...])
  b_ref[...] = jax.nn.sigmoid(
      jnp.dot(g, wb_ref[...], preferred_element_type=jnp.float32) + bb_ref[...])


_k8 = pl.pallas_call(
    _k8_body,
    out_shape=[jax.ShapeDtypeStruct((G, 1), jnp.float32),
               jax.ShapeDtypeStruct((G, 1), jnp.float32)],
)


def kernel(x, edge_index, edge_weight, batch, W1, b1, W2, b2, Wo, bo, Wb, bb):
  src = jnp.pad(edge_index[0], (0, EPAD - E))
  dst = jnp.pad(edge_index[1], (0, EPAD - E))
  eww = jnp.pad(edge_weight, (0, EPAD - E))
  xpad = jnp.pad(x, ((0, ROWPAD - N), (0, HH - FEAT)))
  batch_pad = jnp.pad(batch, (0, ROWPAD - N))
  w1pad = jnp.pad(W1, ((0, HH - FEAT), (0, 0)))

  deg_flat, cnt_flat = _k1(dst, eww, batch_pad)
  dinvb, g0 = _k2(deg_flat.reshape(2, ROWPAD, 1), xpad)
  p1 = _k3(g0, src, dst, eww)
  g1 = _k4(p1.reshape(2, ROWPAD, HH), g0, dinvb, w1pad, b1.reshape(1, HID))
  p2 = _k5(g1.reshape(2 * ROWPAD, HH), src, dst, eww)
  h2 = _k6(p2.reshape(2, ROWPAD, HH), g1, dinvb, W2, b2.reshape(1, HID))
  pool = _k7(h2, batch_pad)
  orange, blue = _k8(pool, cnt_flat.reshape(2, G), Wo, bo.reshape(1, 1),
                     Wb, bb.reshape(1, 1))
  return (orange, blue)
